# baseline (speedup 1.0000x reference)
import functools

import jax
import jax.numpy as jnp
from jax import lax
from jax.experimental import pallas as pl
from jax.experimental.pallas import tpu as pltpu
from jax.experimental.pallas import tpu_sc as plsc

B = 16384
F = 32
U = 1000000
NC, NS = 2, 16
NW = NC * NS
BPW = B // NW
GRP = 16
HGRP = GRP // 2
NGRP = BPW // GRP
LP = 128
BLK = 2048


def _sc_gather_body(bpw, uidx_hbm, iidx_hbm, ug_t, ig_t, um_t, im_t,
                    out_ug, out_ig, out_um, out_im,
                    uidx_v, iidx_v, colbuf, staging, sem0, sem1):
    ngrp = bpw // GRP
    sems = (sem0, sem1)
    wid = lax.axis_index("s") * NC + lax.axis_index("c")
    base = pl.multiple_of(wid * bpw, bpw)
    pltpu.sync_copy(uidx_hbm.at[pl.ds(base, bpw)], uidx_v)
    pltpu.sync_copy(iidx_hbm.at[pl.ds(base, bpw)], iidx_v)

    iota16 = lax.iota(jnp.int32, 16)

    for tab, idx_v, out in (
        (ug_t, uidx_v, out_ug),
        (ig_t, iidx_v, out_ig),
        (um_t, uidx_v, out_um),
        (im_t, iidx_v, out_im),
    ):
        def fire(h, par, tab, idx_v):
            hvec = idx_v[pl.ds(pl.multiple_of((h // 2) * GRP, GRP), GRP)]
            for j in range(HGRP):
                c = hvec[par * HGRP + j]
                al = pl.multiple_of((c >> 7) * 128, 128)
                off = par * HGRP + j
                pltpu.async_copy(tab.at[:, pl.ds(al, 128)],
                                 colbuf.at[:, pl.ds(off * 128, 128)],
                                 sems[par])

        def extract(h, par, tab, idx_v, out):
            pltpu.make_async_copy(
                tab.at[:, pl.ds(0, HGRP * 128)],
                colbuf.at[:, pl.ds(par * HGRP * 128, HGRP * 128)],
                sems[par]).wait()
            lane = lax.rem(iota16, HGRP)
            hb = pl.multiple_of(h * HGRP, HGRP)
            ivec = plsc.load_gather(idx_v, [hb + lane])
            colpos = (par * HGRP + lane) * 128 + lax.rem(ivec, 128)
            for f in range(F):
                fvec = jnp.full((16,), f, jnp.int32)
                vals = plsc.load_gather(colbuf, [fvec, colpos])
                plsc.store_scatter(staging, [lane, fvec], vals,
                                   mask=iota16 < HGRP)
            orow = pl.multiple_of(base + h * HGRP, HGRP)
            pltpu.sync_copy(staging, out.at[pl.ds(orow, HGRP)])

        def step(g, _, tab=tab, idx_v=idx_v, out=out):
            fire(2 * g + 1, 1, tab, idx_v)
            extract(2 * g, 0, tab, idx_v, out)
            fire(2 * g + 2, 0, tab, idx_v)
            extract(2 * g + 1, 1, tab, idx_v, out)
            return 0

        fire(0, 0, tab, idx_v)
        lax.fori_loop(0, ngrp - 1, step, 0)
        fire(2 * (ngrp - 1) + 1, 1, tab, idx_v)
        extract(2 * (ngrp - 1), 0, tab, idx_v, out)
        extract(2 * (ngrp - 1) + 1, 1, tab, idx_v, out)


@functools.cache
def _make_sc_gather(nb):
  bpw = nb // NW
  return pl.kernel(
    functools.partial(_sc_gather_body, bpw),
    out_type=(
        jax.ShapeDtypeStruct((nb, LP), jnp.float32),
        jax.ShapeDtypeStruct((nb, LP), jnp.float32),
        jax.ShapeDtypeStruct((nb, LP), jnp.float32),
        jax.ShapeDtypeStruct((nb, LP), jnp.float32),
    ),
    mesh=plsc.VectorSubcoreMesh(
        core_axis_name="c", subcore_axis_name="s",
        num_cores=NC, num_subcores=NS),
    scratch_types=[
        pltpu.VMEM((bpw,), jnp.int32),
        pltpu.VMEM((bpw,), jnp.int32),
        pltpu.VMEM((F, GRP * 128), jnp.float32),
        pltpu.VMEM((HGRP, LP), jnp.float32),
        pltpu.SemaphoreType.DMA,
        pltpu.SemaphoreType.DMA,
    ],
    compiler_params=pltpu.CompilerParams(
        use_tc_tiling_on_sc=True, needs_layout_passes=False),
  )


def _tc_dense_body(ug, ig, ue, ie,
                   wgT, w0aT, w0bT, b0, w1T, b1, w2T, b2,
                   m0aT, m0bT, mb0, m1T, mb1, m2, mb2, out):
    x = ug[:, :F] * ig[:, :F]
    g = jnp.dot(x, wgT[...], preferred_element_type=jnp.float32)
    v = jnp.maximum(
        jnp.dot(ue[:, :F], w0aT[...], preferred_element_type=jnp.float32)
        + jnp.dot(ie[:, :F], w0bT[...], preferred_element_type=jnp.float32)
        + b0[...], 0.0)
    v = jnp.maximum(
        jnp.dot(v, w1T[...], preferred_element_type=jnp.float32) + b1[...], 0.0)
    v = jnp.dot(v, w2T[...], preferred_element_type=jnp.float32) + b2[...]
    h = jnp.maximum(
        jnp.dot(g, m0aT[...], preferred_element_type=jnp.float32)
        + jnp.dot(v, m0bT[...], preferred_element_type=jnp.float32)
        + mb0[...], 0.0)
    h = jnp.maximum(
        jnp.dot(h, m1T[...], preferred_element_type=jnp.float32) + mb1[...], 0.0)
    o = jnp.sum(h * m2[...], axis=1) + mb2[0, 0]
    out[...] = 1.0 / (1.0 + jnp.exp(-o))


def _full_spec(shape):
    nd = len(shape)
    return pl.BlockSpec(shape, lambda i, _nd=nd: (0,) * _nd)


def _make_dense(nb, wshapes):
    in_specs = [pl.BlockSpec((BLK, LP), lambda i: (i, 0)) for _ in range(4)]
    in_specs += [_full_spec(s) for s in wshapes]
    return pl.pallas_call(
        _tc_dense_body,
        grid=(nb // BLK,),
        in_specs=in_specs,
        out_specs=pl.BlockSpec((BLK,), lambda i: (i,)),
        out_shape=jax.ShapeDtypeStruct((nb,), jnp.float32),
    )


def kernel(user_indices, item_indices, emb_user_gmf, emb_item_gmf, W_gmf,
           emb_user_mlp, emb_item_mlp,
           fc0_W, fc0_b, fc1_W, fc1_b, fc2_W, fc2_b,
           m0_W, m0_b, m1_W, m1_b, m2_W, m2_b):
    ui = user_indices.astype(jnp.int32)
    ii = item_indices.astype(jnp.int32)
    weights = (
        W_gmf.T,
        fc0_W[:, :F].T,
        fc0_W[:, F:].T,
        fc0_b.reshape(1, -1),
        fc1_W.T,
        fc1_b.reshape(1, -1),
        fc2_W.T,
        fc2_b.reshape(1, -1),
        m0_W.T[:8],
        m0_W.T[8:],
        m0_b.reshape(1, -1),
        m1_W.T,
        m1_b.reshape(1, -1),
        m2_W,
        m2_b.reshape(1, 1),
    )
    nb = B // 2
    gather = _make_sc_gather(nb)
    dense = _make_dense(nb, tuple(w.shape for w in weights))
    tabs = (emb_user_gmf.T, emb_item_gmf.T, emb_user_mlp.T, emb_item_mlp.T)
    outs = []
    for c in range(2):
        sl = slice(c * nb, (c + 1) * nb)
        ug, ig, um, im = gather(ui[sl], ii[sl], *tabs)
        outs.append(dense(ug, ig, um, im, *weights))
    return jnp.concatenate(outs)


# speedup vs baseline: 1.0031x; 1.0031x over previous
import functools

import jax
import jax.numpy as jnp
from jax import lax
from jax.experimental import pallas as pl
from jax.experimental.pallas import tpu as pltpu
from jax.experimental.pallas import tpu_sc as plsc

B = 16384
F = 32
U = 1000000
NC, NS = 2, 16
NW = NC * NS
BPW = B // NW
GRP = 16
HGRP = GRP // 2
NGRP = BPW // GRP
LP = 128
BLK = 2048


def _sc_gather_body(uidx_hbm, iidx_hbm, ug_t, ig_t, um_t, im_t,
                    out_ug, out_ig, out_um, out_im,
                    uidx_v, iidx_v, colbuf, staging, sem0, sem1):
    sems = (sem0, sem1)
    wid = lax.axis_index("s") * NC + lax.axis_index("c")
    base = pl.multiple_of(wid * BPW, BPW)
    pltpu.sync_copy(uidx_hbm.at[pl.ds(base, BPW)], uidx_v)
    pltpu.sync_copy(iidx_hbm.at[pl.ds(base, BPW)], iidx_v)

    iota16 = lax.iota(jnp.int32, 16)

    for tab, idx_v, out in (
        (ug_t, uidx_v, out_ug),
        (ig_t, iidx_v, out_ig),
        (um_t, uidx_v, out_um),
        (im_t, iidx_v, out_im),
    ):
        def fire(h, par, tab, idx_v):
            hvec = idx_v[pl.ds(pl.multiple_of((h // 2) * GRP, GRP), GRP)]
            for j in range(HGRP):
                c = hvec[par * HGRP + j]
                al = pl.multiple_of((c >> 7) * 128, 128)
                off = par * HGRP + j
                pltpu.async_copy(tab.at[:, pl.ds(al, 128)],
                                 colbuf.at[:, pl.ds(off * 128, 128)],
                                 sems[par])

        def extract(h, par, tab, idx_v, out):
            pltpu.make_async_copy(
                tab.at[:, pl.ds(0, HGRP * 128)],
                colbuf.at[:, pl.ds(par * HGRP * 128, HGRP * 128)],
                sems[par]).wait()
            lane = lax.rem(iota16, HGRP)
            hb = pl.multiple_of(h * HGRP, HGRP)
            ivec = plsc.load_gather(idx_v, [hb + lane])
            colpos = (par * HGRP + lane) * 128 + lax.rem(ivec, 128)
            for f in range(F):
                fvec = jnp.full((16,), f, jnp.int32)
                vals = plsc.load_gather(colbuf, [fvec, colpos])
                plsc.store_scatter(staging, [lane, fvec], vals,
                                   mask=iota16 < HGRP)
            orow = pl.multiple_of(base + h * HGRP, HGRP)
            pltpu.sync_copy(staging, out.at[pl.ds(orow, HGRP)])

        def step(g, _, tab=tab, idx_v=idx_v, out=out):
            fire(2 * g + 1, 1, tab, idx_v)
            extract(2 * g, 0, tab, idx_v, out)
            fire(2 * g + 2, 0, tab, idx_v)
            extract(2 * g + 1, 1, tab, idx_v, out)
            return 0

        fire(0, 0, tab, idx_v)
        lax.fori_loop(0, NGRP - 1, step, 0)
        fire(2 * (NGRP - 1) + 1, 1, tab, idx_v)
        extract(2 * (NGRP - 1), 0, tab, idx_v, out)
        extract(2 * (NGRP - 1) + 1, 1, tab, idx_v, out)


@functools.cache
def _make_sc_gather():
  return pl.kernel(
    _sc_gather_body,
    out_type=(
        jax.ShapeDtypeStruct((B, LP), jnp.float32),
        jax.ShapeDtypeStruct((B, LP), jnp.float32),
        jax.ShapeDtypeStruct((B, LP), jnp.float32),
        jax.ShapeDtypeStruct((B, LP), jnp.float32),
    ),
    mesh=plsc.VectorSubcoreMesh(
        core_axis_name="c", subcore_axis_name="s",
        num_cores=NC, num_subcores=NS),
    scratch_types=[
        pltpu.VMEM((BPW,), jnp.int32),
        pltpu.VMEM((BPW,), jnp.int32),
        pltpu.VMEM((F, GRP * 128), jnp.float32),
        pltpu.VMEM((HGRP, LP), jnp.float32),
        pltpu.SemaphoreType.DMA,
        pltpu.SemaphoreType.DMA,
    ],
    compiler_params=pltpu.CompilerParams(
        use_tc_tiling_on_sc=True, needs_layout_passes=False,
        disable_bounds_checks=True, disable_semaphore_checks=True),
  )


def _tc_dense_body(ug, ig, ue, ie,
                   wgT, w0aT, w0bT, b0, w1T, b1, w2T, b2,
                   m0aT, m0bT, mb0, m1T, mb1, m2, mb2, out):
    x = ug[:, :F] * ig[:, :F]
    g = jnp.dot(x, wgT[...], preferred_element_type=jnp.float32)
    v = jnp.maximum(
        jnp.dot(ue[:, :F], w0aT[...], preferred_element_type=jnp.float32)
        + jnp.dot(ie[:, :F], w0bT[...], preferred_element_type=jnp.float32)
        + b0[...], 0.0)
    v = jnp.maximum(
        jnp.dot(v, w1T[...], preferred_element_type=jnp.float32) + b1[...], 0.0)
    v = jnp.dot(v, w2T[...], preferred_element_type=jnp.float32) + b2[...]
    h = jnp.maximum(
        jnp.dot(g, m0aT[...], preferred_element_type=jnp.float32)
        + jnp.dot(v, m0bT[...], preferred_element_type=jnp.float32)
        + mb0[...], 0.0)
    h = jnp.maximum(
        jnp.dot(h, m1T[...], preferred_element_type=jnp.float32) + mb1[...], 0.0)
    o = jnp.sum(h * m2[...], axis=1) + mb2[0, 0]
    out[...] = 1.0 / (1.0 + jnp.exp(-o))


def _full_spec(shape):
    nd = len(shape)
    return pl.BlockSpec(shape, lambda i, _nd=nd: (0,) * _nd)


def _make_dense(wshapes):
    in_specs = [pl.BlockSpec((BLK, LP), lambda i: (i, 0)) for _ in range(4)]
    in_specs += [_full_spec(s) for s in wshapes]
    return pl.pallas_call(
        _tc_dense_body,
        grid=(B // BLK,),
        in_specs=in_specs,
        out_specs=pl.BlockSpec((BLK,), lambda i: (i,)),
        out_shape=jax.ShapeDtypeStruct((B,), jnp.float32),
    )


def kernel(user_indices, item_indices, emb_user_gmf, emb_item_gmf, W_gmf,
           emb_user_mlp, emb_item_mlp,
           fc0_W, fc0_b, fc1_W, fc1_b, fc2_W, fc2_b,
           m0_W, m0_b, m1_W, m1_b, m2_W, m2_b):
    ui = user_indices.astype(jnp.int32)
    ii = item_indices.astype(jnp.int32)
    ug, ig, um, im = _make_sc_gather()(
        ui, ii, emb_user_gmf.T, emb_item_gmf.T,
        emb_user_mlp.T, emb_item_mlp.T)
    weights = (
        W_gmf.T,
        fc0_W[:, :F].T,
        fc0_W[:, F:].T,
        fc0_b.reshape(1, -1),
        fc1_W.T,
        fc1_b.reshape(1, -1),
        fc2_W.T,
        fc2_b.reshape(1, -1),
        m0_W.T[:8],
        m0_W.T[8:],
        m0_b.reshape(1, -1),
        m1_W.T,
        m1_b.reshape(1, -1),
        m2_W,
        m2_b.reshape(1, 1),
    )
    dense = _make_dense(tuple(w.shape for w in weights))
    return dense(ug, ig, um, im, *weights)
